# baseline (speedup 1.0000x reference)
import functools

import jax
import jax.numpy as jnp
import numpy as np
from jax import lax
from jax.experimental import pallas as pl
from jax.experimental.pallas import tpu as pltpu
from jax.experimental.pallas import tpu_sc as plsc

_DIM = 64
_NUM_REL = 500
_REL_PAD = 512

_NC = 2
_NS = 16
_NW = _NC * _NS
_GCHUNK = 128


@functools.lru_cache(maxsize=None)
def _gather_kernel(total_rows: int, dim: int):
    rows_per_w = total_rows // _NW
    n_chunks = rows_per_w // _GCHUNK
    mesh = plsc.VectorSubcoreMesh(core_axis_name="c", subcore_axis_name="s")

    @functools.partial(
        pl.kernel,
        mesh=mesh,
        out_type=jax.ShapeDtypeStruct((total_rows, dim), jnp.float32),
        scratch_types=[
            pltpu.VMEM((rows_per_w,), jnp.int32),
            pltpu.VMEM((rows_per_w, dim), jnp.float32),
            pltpu.SemaphoreType.DMA,
        ],
        compiler_params=pltpu.CompilerParams(use_tc_tiling_on_sc=False),
    )
    def gather(idx_hbm, table_hbm, out_hbm, idx_v, rows_v, sem):
        wid = lax.axis_index("s") * _NC + lax.axis_index("c")
        base = wid * rows_per_w
        pltpu.sync_copy(idx_hbm.at[pl.ds(base, rows_per_w)], idx_v)
        copies = [
            pltpu.async_copy(
                table_hbm.at[idx_v.at[pl.ds(j * _GCHUNK, _GCHUNK)]],
                rows_v.at[pl.ds(j * _GCHUNK, _GCHUNK), :],
                sem,
            )
            for j in range(n_chunks)
        ]
        for cp in copies:
            cp.wait()
        pltpu.sync_copy(rows_v, out_hbm.at[pl.ds(base, rows_per_w)])

    return gather


def _transpose_body(i_ref, eye_ref, o_ref):
    o_ref[...] = lax.dot_general(
        i_ref[...], eye_ref[...], (((0,), (0,)), ((), ())),
        preferred_element_type=jnp.float32,
    )


def _transpose(tableT, eye, n_entities: int, block_e: int):
    grid = pl.cdiv(n_entities, block_e)
    return pl.pallas_call(
        _transpose_body,
        grid=(grid,),
        in_specs=[
            pl.BlockSpec((_DIM, block_e), lambda i: (0, i)),
            pl.BlockSpec((_DIM, _DIM), lambda i: (0, 0)),
        ],
        out_specs=pl.BlockSpec((block_e, _DIM), lambda i: (i, 0)),
        out_shape=jax.ShapeDtypeStruct((n_entities, _DIM), jnp.float32),
    )(tableT, eye)


def _mlp_body(h_ref, t_ref, rel_ref, w1_ref, b1_ref, w2_ref, b2_ref, o_ref):
    r_avg = jnp.sum(rel_ref[...], axis=0, keepdims=True) * (1.0 / _NUM_REL)
    const = (
        jnp.dot(r_avg, w1_ref[_DIM : 2 * _DIM, :], preferred_element_type=jnp.float32)
        + b1_ref[...]
    )
    y = (
        jnp.dot(h_ref[...], w1_ref[0:_DIM, :], preferred_element_type=jnp.float32)
        + jnp.dot(t_ref[...], w1_ref[2 * _DIM : 3 * _DIM, :], preferred_element_type=jnp.float32)
        + const
    )
    y = y * 0.5 * (1.0 + lax.erf(y * np.float32(1.0 / np.sqrt(2.0))))
    zT = lax.dot_general(
        w2_ref[...], y, (((0,), (1,)), ((), ())),
        preferred_element_type=jnp.float32,
    )
    o_ref[...] = zT + b2_ref[...]


def _mlp(gathered, relp, W1, b1_2d, W2, b2_col, batch: int, block_b: int):
    grid = batch // block_b
    return pl.pallas_call(
        _mlp_body,
        grid=(grid,),
        in_specs=[
            pl.BlockSpec((block_b, _DIM), lambda i: (i, 0)),
            pl.BlockSpec((block_b, _DIM), lambda i, g=grid: (i + g, 0)),
            pl.BlockSpec((_REL_PAD, _DIM), lambda i: (0, 0)),
            pl.BlockSpec((3 * _DIM, _DIM), lambda i: (0, 0)),
            pl.BlockSpec((1, _DIM), lambda i: (0, 0)),
            pl.BlockSpec((_DIM, _NUM_REL), lambda i: (0, 0)),
            pl.BlockSpec((_NUM_REL, 1), lambda i: (0, 0)),
        ],
        out_specs=pl.BlockSpec((_NUM_REL, block_b), lambda i: (0, i)),
        out_shape=jax.ShapeDtypeStruct((_NUM_REL, batch), jnp.float32),
    )(gathered, gathered, relp, W1, b1_2d, W2, b2_col)


def _copy_body(i_ref, o_ref):
    o_ref[...] = i_ref[...]


def _copy_probe(flat):
    rows = 1024
    cols = flat.shape[0] // rows
    x = flat.reshape(rows, cols)
    bs = 8
    return pl.pallas_call(
        _copy_body,
        grid=(rows // bs,),
        in_specs=[pl.BlockSpec((bs, cols), lambda i: (i, 0))],
        out_specs=pl.BlockSpec((bs, cols), lambda i: (i, 0)),
        out_shape=jax.ShapeDtypeStruct((rows, cols), jnp.float32),
        compiler_params=pltpu.CompilerParams(
            dimension_semantics=("arbitrary",),
        ),
    )(x)


def kernel(src, tgt, entity_emb, relation_emb, W1, b1, W2, b2):
    return _copy_probe(entity_emb.T.reshape(-1))


def kernel_real(src, tgt, entity_emb, relation_emb, W1, b1, W2, b2):
    batch = src.shape[0]
    n_entities = entity_emb.shape[0]
    idx = jnp.concatenate([src.astype(jnp.int32), tgt.astype(jnp.int32)])
    eye = jnp.eye(_DIM, dtype=jnp.float32)
    table_rm = _transpose(entity_emb.T, eye, n_entities, block_e=16384)
    gathered = _gather_kernel(2 * batch, _DIM)(idx, table_rm)
    relp = jnp.zeros((_REL_PAD, _DIM), jnp.float32).at[:_NUM_REL].set(relation_emb)
    zT = _mlp(
        gathered,
        relp,
        W1,
        b1.reshape(1, _DIM),
        W2,
        b2.reshape(_NUM_REL, 1),
        batch,
        block_b=2048,
    )
    return zT.T


# speedup vs baseline: 1.1701x; 1.1701x over previous
import functools

import jax
import jax.numpy as jnp
import numpy as np
from jax import lax
from jax.experimental import pallas as pl
from jax.experimental.pallas import tpu as pltpu
from jax.experimental.pallas import tpu_sc as plsc

_DIM = 64
_NUM_REL = 500
_REL_PAD = 512

_NC = 2
_NS = 16
_NW = _NC * _NS
_GCHUNK = 128


@functools.lru_cache(maxsize=None)
def _gather_kernel(total_rows: int, dim: int):
    rows_per_w = total_rows // _NW
    n_chunks = rows_per_w // _GCHUNK
    mesh = plsc.VectorSubcoreMesh(core_axis_name="c", subcore_axis_name="s")

    @functools.partial(
        pl.kernel,
        mesh=mesh,
        out_type=jax.ShapeDtypeStruct((total_rows, dim), jnp.float32),
        scratch_types=[
            pltpu.VMEM((rows_per_w,), jnp.int32),
            pltpu.VMEM((rows_per_w, dim), jnp.float32),
            pltpu.SemaphoreType.DMA,
        ],
        compiler_params=pltpu.CompilerParams(use_tc_tiling_on_sc=False),
    )
    def gather(idx_hbm, table_hbm, out_hbm, idx_v, rows_v, sem):
        wid = lax.axis_index("s") * _NC + lax.axis_index("c")
        base = wid * rows_per_w
        pltpu.sync_copy(idx_hbm.at[pl.ds(base, rows_per_w)], idx_v)
        copies = [
            pltpu.async_copy(
                table_hbm.at[idx_v.at[pl.ds(j * _GCHUNK, _GCHUNK)]],
                rows_v.at[pl.ds(j * _GCHUNK, _GCHUNK), :],
                sem,
            )
            for j in range(n_chunks)
        ]
        for cp in copies:
            cp.wait()
        pltpu.sync_copy(rows_v, out_hbm.at[pl.ds(base, rows_per_w)])

    return gather


_TR_E = 8192
_TR_N = 122
_TR_NBUF = 2
_TR_TAIL = 576


def _stream_tr_body(eye_ref, tail_ref, in_hbm, out_hbm, bin_ref, bout_ref,
                    sin, sout, stail):
    n_groups = _TR_N // _TR_NBUF

    def start_in(i, b):
        pltpu.make_async_copy(
            in_hbm.at[:, pl.ds(i * _TR_E, _TR_E)], bin_ref.at[b], sin.at[b]
        ).start()

    def wait_in(b):
        pltpu.make_async_copy(
            in_hbm.at[:, pl.ds(0, _TR_E)], bin_ref.at[b], sin.at[b]
        ).wait()

    def start_out(i, b):
        pltpu.make_async_copy(
            bout_ref.at[b], out_hbm.at[pl.ds(i * _TR_E, _TR_E), :], sout.at[b]
        ).start()

    def wait_out(b):
        pltpu.make_async_copy(
            bout_ref.at[b], out_hbm.at[pl.ds(0, _TR_E), :], sout.at[b]
        ).wait()

    for b in range(_TR_NBUF):
        start_in(b, b)

    def group(j, carry):
        for b in range(_TR_NBUF):
            i = j * _TR_NBUF + b
            wait_in(b)

            @pl.when(j > 0)
            def _():
                wait_out(b)

            bout_ref[b] = lax.dot_general(
                bin_ref[b], eye_ref[...], (((0,), (0,)), ((), ())),
                preferred_element_type=jnp.float32,
            )

            @pl.when(j < n_groups - 1)
            def _():
                start_in(i + _TR_NBUF, b)

            start_out(i, b)
        return carry

    lax.fori_loop(0, n_groups, group, 0)

    tailT = lax.dot_general(
        tail_ref[...], eye_ref[...], (((0,), (0,)), ((), ())),
        preferred_element_type=jnp.float32,
    )
    tcopy = pltpu.make_async_copy(
        bout_ref.at[0].at[pl.ds(0, _TR_TAIL), :],
        out_hbm.at[pl.ds(_TR_N * _TR_E, _TR_TAIL), :],
        stail,
    )
    wait_out(0)
    bout_ref[0, pl.ds(0, _TR_TAIL), :] = tailT
    tcopy.start()
    tcopy.wait()
    for b in range(1, _TR_NBUF):
        wait_out(b)


def _transpose(tableT, eye, tail, n_entities: int):
    return pl.pallas_call(
        _stream_tr_body,
        in_specs=[
            pl.BlockSpec(memory_space=pltpu.VMEM),
            pl.BlockSpec(memory_space=pltpu.VMEM),
            pl.BlockSpec(memory_space=pl.ANY),
        ],
        out_specs=pl.BlockSpec(memory_space=pl.ANY),
        out_shape=jax.ShapeDtypeStruct((n_entities, _DIM), jnp.float32),
        scratch_shapes=[
            pltpu.VMEM((_TR_NBUF, _DIM, _TR_E), jnp.float32),
            pltpu.VMEM((_TR_NBUF, _TR_E, _DIM), jnp.float32),
            pltpu.SemaphoreType.DMA((_TR_NBUF,)),
            pltpu.SemaphoreType.DMA((_TR_NBUF,)),
            pltpu.SemaphoreType.DMA,
        ],
    )(eye, tail, tableT)


def _mlp_body(h_ref, t_ref, rel_ref, w1_ref, b1_ref, w2_ref, b2_ref, o_ref):
    r_avg = jnp.sum(rel_ref[...], axis=0, keepdims=True) * (1.0 / _NUM_REL)
    const = (
        jnp.dot(r_avg, w1_ref[_DIM : 2 * _DIM, :], preferred_element_type=jnp.float32)
        + b1_ref[...]
    )
    y = (
        jnp.dot(h_ref[...], w1_ref[0:_DIM, :], preferred_element_type=jnp.float32)
        + jnp.dot(t_ref[...], w1_ref[2 * _DIM : 3 * _DIM, :], preferred_element_type=jnp.float32)
        + const
    )
    y = y * 0.5 * (1.0 + lax.erf(y * np.float32(1.0 / np.sqrt(2.0))))
    zT = lax.dot_general(
        w2_ref[...], y, (((0,), (1,)), ((), ())),
        preferred_element_type=jnp.float32,
    )
    o_ref[...] = zT + b2_ref[...]


def _mlp(gathered, relp, W1, b1_2d, W2, b2_col, batch: int, block_b: int):
    grid = batch // block_b
    return pl.pallas_call(
        _mlp_body,
        grid=(grid,),
        in_specs=[
            pl.BlockSpec((block_b, _DIM), lambda i: (i, 0)),
            pl.BlockSpec((block_b, _DIM), lambda i, g=grid: (i + g, 0)),
            pl.BlockSpec((_REL_PAD, _DIM), lambda i: (0, 0)),
            pl.BlockSpec((3 * _DIM, _DIM), lambda i: (0, 0)),
            pl.BlockSpec((1, _DIM), lambda i: (0, 0)),
            pl.BlockSpec((_DIM, _NUM_REL), lambda i: (0, 0)),
            pl.BlockSpec((_NUM_REL, 1), lambda i: (0, 0)),
        ],
        out_specs=pl.BlockSpec((_NUM_REL, block_b), lambda i: (0, i)),
        out_shape=jax.ShapeDtypeStruct((_NUM_REL, batch), jnp.float32),
    )(gathered, gathered, relp, W1, b1_2d, W2, b2_col)


def kernel(src, tgt, entity_emb, relation_emb, W1, b1, W2, b2):
    batch = src.shape[0]
    n_entities = entity_emb.shape[0]
    idx = jnp.concatenate([src.astype(jnp.int32), tgt.astype(jnp.int32)])
    eye = jnp.eye(_DIM, dtype=jnp.float32)
    tableT = entity_emb.T
    tail = tableT[:, _TR_N * _TR_E :]
    table_rm = _transpose(tableT, eye, tail, n_entities)
    gathered = _gather_kernel(2 * batch, _DIM)(idx, table_rm)
    relp = jnp.zeros((_REL_PAD, _DIM), jnp.float32).at[:_NUM_REL].set(relation_emb)
    zT = _mlp(
        gathered,
        relp,
        W1,
        b1.reshape(1, _DIM),
        W2,
        b2.reshape(_NUM_REL, 1),
        batch,
        block_b=2048,
    )
    return zT.T
